# baseline (speedup 1.0000x reference)
import functools

import jax
import jax.numpy as jnp
import numpy as np
from jax.experimental import pallas as pl
from jax.experimental.pallas import tpu as pltpu

_INPUT_SIZE = 128
_T = 16
_V = 64


def _decode_block(x_ref, a_ref, seq_ref, score_ref, *, b, jpb):
    for j in range(jpb):
        xj = x_ref[:, j * _INPUT_SIZE:(j + 1) * _INPUT_SIZE]
        lt = jax.lax.dot_general(
            a_ref[...], xj,
            dimension_numbers=(((1,), (1,)), ((), ())),
            preferred_element_type=jnp.float32,
        )
        l3 = lt.reshape(_T, _V, b)
        maxv = jnp.max(l3, axis=1)
        hit = l3 == maxv[:, None, :]
        vio = jax.lax.broadcasted_iota(jnp.int32, (_T, _V, b), 1).astype(jnp.float32)
        idxf = jnp.min(jnp.where(hit, vio, jnp.float32(_V)), axis=1)
        seq_ref[j] = idxf.astype(jnp.int32)
        score_ref[j, 0, :] = jnp.sum(maxv, axis=0)


def kernel(observation, W):
    batch, d = observation.shape
    ipo = d // _INPUT_SIZE
    a = W.reshape(_INPUT_SIZE, _T * _V).T

    jpb = 4
    br = 256
    grid = (batch // br, ipo // jpb)
    seqs_t, scores_t = pl.pallas_call(
        functools.partial(_decode_block, b=br, jpb=jpb),
        grid=grid,
        compiler_params=pltpu.CompilerParams(
            dimension_semantics=("parallel", "parallel"),
        ),
        in_specs=[
            pl.BlockSpec((br, jpb * _INPUT_SIZE), lambda r, i: (r, i)),
            pl.BlockSpec((_T * _V, _INPUT_SIZE), lambda r, i: (0, 0)),
        ],
        out_specs=[
            pl.BlockSpec((jpb, _T, br), lambda r, i: (i, 0, r)),
            pl.BlockSpec((jpb, 1, br), lambda r, i: (i, 0, r)),
        ],
        out_shape=[
            jax.ShapeDtypeStruct((ipo, _T, batch), jnp.int32),
            jax.ShapeDtypeStruct((ipo, 1, batch), jnp.float32),
        ],
    )(observation, a)

    seq_supp_batch = jnp.transpose(seqs_t, (2, 0, 1))
    score_batch = scores_t.reshape(ipo, batch).T
    length_supp_batch = jnp.full((batch, ipo), _T, dtype=jnp.int32)
    return seq_supp_batch, length_supp_batch, score_batch


# speedup vs baseline: 1.0993x; 1.0993x over previous
import functools

import jax
import jax.numpy as jnp
import numpy as np
from jax.experimental import pallas as pl
from jax.experimental.pallas import tpu as pltpu

_INPUT_SIZE = 128
_T = 16
_V = 64


def _decode_block(x_ref, a_ref, seq_ref, score_ref, *, b, jpb):
    for j in range(jpb):
        xj = x_ref[:, j * _INPUT_SIZE:(j + 1) * _INPUT_SIZE]
        lt = jax.lax.dot_general(
            a_ref[...], xj,
            dimension_numbers=(((1,), (1,)), ((), ())),
            preferred_element_type=jnp.float32,
        )
        l3 = lt.reshape(_T, _V, b)
        maxv = jnp.max(l3, axis=1)
        hit = l3 == maxv[:, None, :]
        vio = jax.lax.broadcasted_iota(jnp.int32, (_T, _V, b), 1).astype(jnp.float32)
        idxf = jnp.min(jnp.where(hit, vio, jnp.float32(_V)), axis=1)
        seq_ref[j] = idxf.astype(jnp.int32)
        score_ref[j, 0, :] = jnp.sum(maxv, axis=0)


def kernel(observation, W):
    batch, d = observation.shape
    ipo = d // _INPUT_SIZE
    a = W.reshape(_INPUT_SIZE, _T * _V).T

    jpb = 4
    grid = (ipo // jpb,)
    seqs_t, scores_t = pl.pallas_call(
        functools.partial(_decode_block, b=batch, jpb=jpb),
        grid=grid,
        compiler_params=pltpu.CompilerParams(
            dimension_semantics=("parallel",),
        ),
        in_specs=[
            pl.BlockSpec((batch, jpb * _INPUT_SIZE), lambda i: (0, i)),
            pl.BlockSpec((_T * _V, _INPUT_SIZE), lambda i: (0, 0)),
        ],
        out_specs=[
            pl.BlockSpec((jpb, _T, batch), lambda i: (i, 0, 0)),
            pl.BlockSpec((jpb, 1, batch), lambda i: (i, 0, 0)),
        ],
        out_shape=[
            jax.ShapeDtypeStruct((ipo, _T, batch), jnp.int32),
            jax.ShapeDtypeStruct((ipo, 1, batch), jnp.float32),
        ],
    )(observation, a)

    seq_supp_batch = jnp.transpose(seqs_t, (2, 0, 1))
    score_batch = scores_t.reshape(ipo, batch).T
    length_supp_batch = jnp.full((batch, ipo), _T, dtype=jnp.int32)
    return seq_supp_batch, length_supp_batch, score_batch
